# Initial kernel scaffold; baseline (speedup 1.0000x reference)
import functools

import jax
import jax.numpy as jnp
from jax import lax
from jax.experimental import pallas as pl
from jax.experimental.pallas import tpu as pltpu
from jax.experimental.pallas import tpu_sc as plsc

VOCAB = 1000000
EMBED_OUT = 32
CONTEXT = 20
N_ACTIONS = 6
L1 = 256
BATCH = 16384

N_IDX = BATCH * CONTEXT
NUM_WORKERS = 32
PER_WORKER = N_IDX // NUM_WORKERS
CHUNK = 2048
N_CHUNKS = PER_WORKER // CHUNK


def _sc_gather(flat_idx, emb_table):
    mesh = plsc.VectorSubcoreMesh(core_axis_name="c", subcore_axis_name="s")

    @functools.partial(
        pl.kernel,
        out_type=jax.ShapeDtypeStruct((N_IDX, EMBED_OUT), jnp.float32),
        mesh=mesh,
        scratch_types=[
            pltpu.VMEM((CHUNK,), jnp.int32),
            pltpu.VMEM((CHUNK, EMBED_OUT), jnp.float32),
            pltpu.SemaphoreType.DMA,
        ],
    )
    def gather_kernel(idx_hbm, table_hbm, out_hbm, idx_v, rows_v, sem):
        wid = lax.axis_index("s") * 2 + lax.axis_index("c")
        base = wid * PER_WORKER
        for c in range(N_CHUNKS):
            off = base + c * CHUNK
            pltpu.sync_copy(idx_hbm.at[pl.ds(off, CHUNK)], idx_v)
            pltpu.async_copy(table_hbm.at[idx_v], rows_v, sem).wait()
            pltpu.sync_copy(rows_v, out_hbm.at[pl.ds(off, CHUNK)])

    return gather_kernel(flat_idx, emb_table)


def _mlp_body(e_ref, w1_ref, b1_ref, w2_ref, b2_ref, o_ref):
    h = jnp.dot(e_ref[...], w1_ref[...], preferred_element_type=jnp.float32)
    h = jnp.maximum(h + b1_ref[...], 0.0)
    o_ref[...] = jnp.dot(h, w2_ref[...], preferred_element_type=jnp.float32) + b2_ref[...]


def _tc_mlp(embeds, W1, b1, W2, b2):
    BM = 2048
    grid = (BATCH // BM,)
    in1 = EMBED_OUT * CONTEXT
    return pl.pallas_call(
        _mlp_body,
        grid=grid,
        in_specs=[
            pl.BlockSpec((BM, in1), lambda i: (i, 0)),
            pl.BlockSpec((in1, L1), lambda i: (0, 0)),
            pl.BlockSpec((1, L1), lambda i: (0, 0)),
            pl.BlockSpec((L1, N_ACTIONS), lambda i: (0, 0)),
            pl.BlockSpec((1, N_ACTIONS), lambda i: (0, 0)),
        ],
        out_specs=pl.BlockSpec((BM, N_ACTIONS), lambda i: (i, 0)),
        out_shape=jax.ShapeDtypeStruct((BATCH, N_ACTIONS), jnp.float32),
    )(embeds, W1, b1.reshape(1, L1), W2, b2.reshape(1, N_ACTIONS))


def kernel(x, emb_table, W1, b1, W2, b2):
    flat_idx = x.reshape(N_IDX)
    rows = _sc_gather(flat_idx, emb_table)
    embeds = rows.reshape(BATCH, EMBED_OUT * CONTEXT)
    return _tc_mlp(embeds, W1, b1, W2, b2)


# speedup vs baseline: 14.1219x; 14.1219x over previous
import functools

import jax
import jax.numpy as jnp
from jax import lax
from jax.experimental import pallas as pl
from jax.experimental.pallas import tpu as pltpu
from jax.experimental.pallas import tpu_sc as plsc

VOCAB = 1000000
EMBED_OUT = 32
CONTEXT = 20
N_ACTIONS = 6
L1 = 256
BATCH = 16384

N_IDX = BATCH * CONTEXT
NUM_WORKERS = 32
PER_WORKER = N_IDX // NUM_WORKERS
CHUNK = 2048
N_CHUNKS = PER_WORKER // CHUNK


def _sc_gather(flat_idx, emb_table):
    mesh = plsc.VectorSubcoreMesh(core_axis_name="c", subcore_axis_name="s")

    @functools.partial(
        pl.kernel,
        out_type=jax.ShapeDtypeStruct((N_IDX, EMBED_OUT), jnp.float32),
        mesh=mesh,
        scratch_types=[
            pltpu.VMEM((CHUNK,), jnp.int32),
            pltpu.VMEM((CHUNK, EMBED_OUT), jnp.float32),
            pltpu.SemaphoreType.DMA,
        ],
        compiler_params=pltpu.CompilerParams(use_tc_tiling_on_sc=False),
    )
    def gather_kernel(idx_hbm, table_hbm, out_hbm, idx_v, rows_v, sem):
        wid = lax.axis_index("s") * 2 + lax.axis_index("c")
        base = wid * PER_WORKER
        for c in range(N_CHUNKS):
            off = base + c * CHUNK
            pltpu.sync_copy(idx_hbm.at[pl.ds(off, CHUNK)], idx_v)
            pltpu.async_copy(table_hbm.at[idx_v], rows_v, sem).wait()
            pltpu.sync_copy(rows_v, out_hbm.at[pl.ds(off, CHUNK)])

    return gather_kernel(flat_idx, emb_table)


def _mlp_body(e_ref, w1_ref, b1_ref, w2_ref, b2_ref, o_ref):
    h = jnp.dot(e_ref[...], w1_ref[...], preferred_element_type=jnp.float32)
    h = jnp.maximum(h + b1_ref[...], 0.0)
    o_ref[...] = jnp.dot(h, w2_ref[...], preferred_element_type=jnp.float32) + b2_ref[...]


def _tc_mlp(embeds, W1, b1, W2, b2):
    BM = 2048
    grid = (BATCH // BM,)
    in1 = EMBED_OUT * CONTEXT
    return pl.pallas_call(
        _mlp_body,
        grid=grid,
        in_specs=[
            pl.BlockSpec((BM, in1), lambda i: (i, 0)),
            pl.BlockSpec((in1, L1), lambda i: (0, 0)),
            pl.BlockSpec((1, L1), lambda i: (0, 0)),
            pl.BlockSpec((L1, N_ACTIONS), lambda i: (0, 0)),
            pl.BlockSpec((1, N_ACTIONS), lambda i: (0, 0)),
        ],
        out_specs=pl.BlockSpec((BM, N_ACTIONS), lambda i: (i, 0)),
        out_shape=jax.ShapeDtypeStruct((BATCH, N_ACTIONS), jnp.float32),
    )(embeds, W1, b1.reshape(1, L1), W2, b2.reshape(1, N_ACTIONS))


def kernel(x, emb_table, W1, b1, W2, b2):
    flat_idx = x.reshape(N_IDX)
    rows = _sc_gather(flat_idx, emb_table)
    embeds = rows.reshape(BATCH, EMBED_OUT * CONTEXT)
    return _tc_mlp(embeds, W1, b1, W2, b2)


# speedup vs baseline: 18.8698x; 1.3362x over previous
import functools

import jax
import jax.numpy as jnp
from jax import lax
from jax.experimental import pallas as pl
from jax.experimental.pallas import tpu as pltpu
from jax.experimental.pallas import tpu_sc as plsc

VOCAB = 1000000
EMBED_OUT = 32
CONTEXT = 20
N_ACTIONS = 6
L1 = 256
BATCH = 16384

N_IDX = BATCH * CONTEXT
NUM_WORKERS = 32
PER_WORKER = N_IDX // NUM_WORKERS
CHUNK = 2048
N_CHUNKS = PER_WORKER // CHUNK

VPAD = 1024000
STRIP = VPAD // 4
BR = 2048
PACK_GRID = STRIP // BR
LAST_BLK = (VOCAB - 1) // BR


def _pack_body(x0_ref, x1_ref, x2_ref, x3_ref, o_ref):
    for q, xq in enumerate((x0_ref, x1_ref, x2_ref, x3_ref)):
        o_ref[:, 32 * q:32 * (q + 1)] = xq[...].T


def _tc_pack(table_t):
    def strip_spec(q):
        return pl.BlockSpec(
            (EMBED_OUT, BR),
            lambda i, q=q: (0, jnp.minimum(PACK_GRID * q + i, LAST_BLK)),
        )
    return pl.pallas_call(
        _pack_body,
        grid=(PACK_GRID,),
        in_specs=[strip_spec(q) for q in range(4)],
        out_specs=pl.BlockSpec((BR, 128), lambda i: (i, 0)),
        out_shape=jax.ShapeDtypeStruct((STRIP, 128), jnp.float32),
    )(table_t, table_t, table_t, table_t)


def _sc_gather(flat_idx, table_lin):
    mesh = plsc.VectorSubcoreMesh(core_axis_name="c", subcore_axis_name="s")

    @functools.partial(
        pl.kernel,
        out_type=jax.ShapeDtypeStruct((N_IDX, EMBED_OUT), jnp.float32),
        mesh=mesh,
        scratch_types=[
            pltpu.VMEM((CHUNK,), jnp.int32),
            pltpu.VMEM((CHUNK, EMBED_OUT), jnp.float32),
            pltpu.SemaphoreType.DMA,
        ],
        compiler_params=pltpu.CompilerParams(use_tc_tiling_on_sc=False),
    )
    def gather_kernel(idx_hbm, table_hbm, out_hbm, idx_v, rows_v, sem):
        wid = lax.axis_index("s") * 2 + lax.axis_index("c")
        base = wid * PER_WORKER
        for c in range(N_CHUNKS):
            off = base + c * CHUNK
            pltpu.sync_copy(idx_hbm.at[pl.ds(off, CHUNK)], idx_v)
            pltpu.async_copy(table_hbm.at[idx_v], rows_v, sem).wait()
            pltpu.sync_copy(rows_v, out_hbm.at[pl.ds(off, CHUNK)])

    return gather_kernel(flat_idx, table_lin)


BM = 2048
N_J = CONTEXT * EMBED_OUT // 128
TILES_PER_BLOCK = BM // 8 * N_J


def _mlp_body(e_ref, w1_ref, b1_ref, w2_ref, b2_ref, o_ref):
    x4 = e_ref[...].reshape(BM // 8, N_J, 8, 128)
    acc = jnp.zeros((BM, L1), jnp.float32)
    for j in range(N_J):
        ej = x4[:, j].reshape(BM, 128)
        acc += jnp.dot(ej, w1_ref[j], preferred_element_type=jnp.float32)
    h = jnp.maximum(acc + b1_ref[...], 0.0)
    o_ref[...] = jnp.dot(h, w2_ref[...], preferred_element_type=jnp.float32) + b2_ref[...]


def _tc_mlp(e3, W1, b1, W2, b2):
    grid = (BATCH // BM,)
    return pl.pallas_call(
        _mlp_body,
        grid=grid,
        in_specs=[
            pl.BlockSpec((TILES_PER_BLOCK, 8, 128), lambda i: (i, 0, 0)),
            pl.BlockSpec((N_J, 128, L1), lambda i: (0, 0, 0)),
            pl.BlockSpec((1, L1), lambda i: (0, 0)),
            pl.BlockSpec((L1, N_ACTIONS), lambda i: (0, 0)),
            pl.BlockSpec((1, N_ACTIONS), lambda i: (0, 0)),
        ],
        out_specs=pl.BlockSpec((BM, N_ACTIONS), lambda i: (i, 0)),
        out_shape=jax.ShapeDtypeStruct((BATCH, N_ACTIONS), jnp.float32),
    )(e3, W1.reshape(N_J, 128, L1), b1.reshape(1, L1),
      W2, b2.reshape(1, N_ACTIONS))


def kernel(x, emb_table, W1, b1, W2, b2):
    packed = _tc_pack(emb_table.T)
    table_lin = packed.reshape(VPAD, EMBED_OUT)
    vidx = (x % STRIP) * 4 + x // STRIP
    pidx = vidx.reshape(BATCH // 8, 8, N_J, 4).transpose(0, 2, 1, 3).reshape(N_IDX)
    rows = _sc_gather(pidx, table_lin)
    e3 = rows.reshape(N_IDX * EMBED_OUT // (8 * 128), 8, 128)
    return _tc_mlp(e3, W1, b1, W2, b2)
